# baseline (speedup 1.0000x reference)
import functools

import jax
import jax.numpy as jnp
from jax import lax
from jax.experimental import pallas as pl
from jax.experimental.pallas import tpu as pltpu
from jax.experimental.pallas import tpu_sc as plsc

_B = 16384
_NSPLIT = 2
_BC = _B // _NSPLIT
_K = 128
_NC = 2
_NS = 16
_NW = _NC * _NS
_BPW = _BC // _NW
_CH = 128
_NCH = _BPW // _CH


def _sc_gather_body(uidx_hbm, iidx_hbm, didx_hbm, utab_hbm, itab_hbm,
                    c0_hbm, c1_hbm, c2_hbm,
                    inter_hbm, bvals_hbm,
                    uidx_v, iidx_v, didx_v, b0_v, b1_v, b2_v, bbuf_v,
                    ubuf_v, ibuf_v,
                    semgu, semgi, semb, sems0, sems1, sembst):
    wid = lax.axis_index("s") * _NC + lax.axis_index("c")
    base = wid * _BPW

    pltpu.sync_copy(uidx_hbm.at[pl.ds(base, _BPW)], uidx_v)
    pltpu.sync_copy(iidx_hbm.at[pl.ds(base, _BPW)], iidx_v)
    pltpu.sync_copy(didx_hbm.at[pl.ds(base, _BPW)], didx_v)

    bcopies = []
    for c in range(_NCH):
        sl = pl.ds(c * _CH, _CH)
        isl = uidx_v.at[sl]
        bcopies.append(pltpu.async_copy(c0_hbm.at[isl], b0_v.at[sl], semb))
        bcopies.append(pltpu.async_copy(c1_hbm.at[isl], b1_v.at[sl], semb))
        bcopies.append(pltpu.async_copy(c2_hbm.at[isl], b2_v.at[sl], semb))

    sems = (sems0, sems1)
    st = [None] * _NCH

    def fire(c):
        b = c % 2
        sl = pl.ds(c * _CH, _CH)
        return (
            pltpu.async_copy(utab_hbm.at[uidx_v.at[sl]], ubuf_v.at[b], semgu),
            pltpu.async_copy(itab_hbm.at[iidx_v.at[sl]], ibuf_v.at[b], semgi),
        )

    g = {0: fire(0)}
    for c in range(_NCH):
        b = c % 2
        g[c][0].wait()
        g[c][1].wait()
        if c + 1 < _NCH:
            if c >= 1:
                st[c - 1].wait()
            g[c + 1] = fire(c + 1)

        def _mul(r, carry):
            for j in range(_K // 16):
                s = pl.ds(j * 16, 16)
                ubuf_v[b, r, s] = ubuf_v[b, r, s] * ibuf_v[b, r, s]
            return carry
        lax.fori_loop(0, _CH, _mul, 0)
        osl = pl.ds(base + c * _CH, _CH)
        st[c] = pltpu.async_copy(ubuf_v.at[b], inter_hbm.at[osl], sems[b])

    for bc in bcopies:
        bc.wait()

    def _sel(j, carry):
        s = pl.ds(j * 16, 16)
        d = didx_v[s]
        bbuf_v[s] = jnp.where(d == 0, b0_v[s],
                              jnp.where(d == 1, b1_v[s], b2_v[s]))
        return carry
    lax.fori_loop(0, _BPW // 16, _sel, 0)
    bst = pltpu.async_copy(bbuf_v, bvals_hbm.at[pl.ds(base, _BPW)], sembst)

    st[_NCH - 2].wait()
    st[_NCH - 1].wait()
    bst.wait()


@functools.cache
def _make_sc_gather():
    return functools.partial(
        pl.kernel,
        out_type=[
            jax.ShapeDtypeStruct((_BC, _K), jnp.float32),
            jax.ShapeDtypeStruct((_BC,), jnp.float32),
        ],
        mesh=plsc.VectorSubcoreMesh(core_axis_name="c", subcore_axis_name="s"),
        scratch_types=[
            pltpu.VMEM((_BPW,), jnp.int32),
            pltpu.VMEM((_BPW,), jnp.int32),
            pltpu.VMEM((_BPW,), jnp.int32),
            pltpu.VMEM((_BPW,), jnp.float32),
            pltpu.VMEM((_BPW,), jnp.float32),
            pltpu.VMEM((_BPW,), jnp.float32),
            pltpu.VMEM((_BPW,), jnp.float32),
            pltpu.VMEM((2, _CH, _K), jnp.float32),
            pltpu.VMEM((2, _CH, _K), jnp.float32),
            pltpu.SemaphoreType.DMA,
            pltpu.SemaphoreType.DMA,
            pltpu.SemaphoreType.DMA,
            pltpu.SemaphoreType.DMA,
            pltpu.SemaphoreType.DMA,
            pltpu.SemaphoreType.DMA,
        ],
    )(_sc_gather_body)


_BLK = 4096


_NCODE = 120


def _tc_mlp_body(inter_ref, code_ref, bias_ref,
                 dt_ref, wk_ref, yr_ref, w1a_ref, w1t_ref, b1_ref,
                 w2_ref, b2_ref, out_ref):
    f32 = jnp.float32
    inter = inter_ref[...]
    w1t = w1t_ref[...]
    pd = jnp.dot(dt_ref[...], w1t[0:10], preferred_element_type=f32)
    pw = jnp.dot(wk_ref[...], w1t[10:20], preferred_element_type=f32)
    py = jnp.dot(yr_ref[...], w1t[20:30], preferred_element_type=f32)
    c0 = lax.broadcasted_iota(jnp.int32, (_NCODE, 1), 0)
    e3 = (c0 % 3 == lax.broadcasted_iota(jnp.int32, (_NCODE, 3), 1)).astype(f32)
    e2 = ((c0 // 3) % 2
          == lax.broadcasted_iota(jnp.int32, (_NCODE, 2), 1)).astype(f32)
    e20 = (c0 // 6
           == lax.broadcasted_iota(jnp.int32, (_NCODE, 20), 1)).astype(f32)
    p = jnp.dot(e3, pd, preferred_element_type=f32) \
        + jnp.dot(e2, pw, preferred_element_type=f32) \
        + jnp.dot(e20, py, preferred_element_type=f32)
    code = code_ref[...]
    hot_t = (jnp.broadcast_to(code, (_NCODE, _BLK))
             == lax.broadcasted_iota(jnp.int32, (_NCODE, _BLK), 0)).astype(f32)
    acc = jnp.dot(inter, w1a_ref[...], preferred_element_type=f32)
    acc = acc + lax.dot_general(hot_t, p, (((0,), (0,)), ((), ())),
                                preferred_element_type=f32)
    h = jnp.maximum(acc + b1_ref[...], 0.0)
    w2 = w2_ref[...]
    parts = [
        lax.dot_general(w2, h[k * 128:(k + 1) * 128, :],
                        (((0,), (1,)), ((), ())), preferred_element_type=f32)
        for k in range(_BLK // 128)
    ]
    out_t = jnp.concatenate(parts, axis=0)
    out_ref[...] = out_t.reshape(_BLK) + b2_ref[0, 0] + bias_ref[...]


def _tc_mlp(inter, code, bvals, dt, wk, yr, w1a, w1t, b1, w2, b2):
    grid = (_BC // _BLK,)
    row_spec = pl.BlockSpec((_BLK, _K), lambda i: (i, 0))
    vec_spec = pl.BlockSpec((_BLK,), lambda i: (i,))

    def full(a):
        return pl.BlockSpec(a.shape, lambda i: tuple(0 for _ in a.shape))

    return pl.pallas_call(
        _tc_mlp_body,
        grid=grid,
        in_specs=[row_spec, vec_spec, vec_spec,
                  full(dt), full(wk), full(yr), full(w1a), full(w1t),
                  full(b1), full(w2), full(b2)],
        out_specs=vec_spec,
        out_shape=jax.ShapeDtypeStruct((_BC,), jnp.float32),
    )(inter, code, bvals, dt, wk, yr, w1a, w1t, b1, w2, b2)


def kernel(user_input, item_input, daytime_input, weekend_input, year_input,
           user_table, item_table, daytime_table, weekend_table, year_table,
           user_time_bias, W1, b1, W2, b2):
    ui = user_input.astype(jnp.int32)
    ii = item_input.astype(jnp.int32)
    di = daytime_input.astype(jnp.int32)
    code = di + 3 * weekend_input.astype(jnp.int32) \
        + 6 * year_input.astype(jnp.int32)
    c0 = user_time_bias[:, 0]
    c1 = user_time_bias[:, 1]
    c2 = user_time_bias[:, 2]
    w1a = W1[0:_K]
    w1t = W1[_K:]
    b1r = b1.reshape(1, -1)
    b2r = b2.reshape(1, 1)
    outs = []
    for h in range(_NSPLIT):
        sl = slice(h * _BC, (h + 1) * _BC)
        inter, bvals = _make_sc_gather()(
            ui[sl], ii[sl], di[sl], user_table, item_table, c0, c1, c2)
        outs.append(_tc_mlp(
            inter, code[sl], bvals,
            daytime_table, weekend_table, year_table,
            w1a, w1t, b1r, W2, b2r))
    return jnp.concatenate(outs)


# speedup vs baseline: 1.0734x; 1.0734x over previous
import functools

import jax
import jax.numpy as jnp
from jax import lax
from jax.experimental import pallas as pl
from jax.experimental.pallas import tpu as pltpu
from jax.experimental.pallas import tpu_sc as plsc

_B = 16384
_K = 128
_NC = 2
_NS = 16
_NW = _NC * _NS
_BPW = _B // _NW
_CH = 128
_NCH = _BPW // _CH


def _sc_gather_body(uidx_hbm, iidx_hbm, didx_hbm, utab_hbm, itab_hbm,
                    c0_hbm, c1_hbm, c2_hbm,
                    inter_hbm, bvals_hbm,
                    uidx_v, iidx_v, didx_v, b0_v, b1_v, b2_v, bbuf_v,
                    ubuf_v, ibuf_v,
                    semgu, semgi, semb, sems0, sems1, sembst):
    wid = lax.axis_index("s") * _NC + lax.axis_index("c")
    base = wid * _BPW

    pltpu.sync_copy(uidx_hbm.at[pl.ds(base, _BPW)], uidx_v)
    pltpu.sync_copy(iidx_hbm.at[pl.ds(base, _BPW)], iidx_v)
    pltpu.sync_copy(didx_hbm.at[pl.ds(base, _BPW)], didx_v)

    def _bgather(c, carry):
        sl = pl.ds(c * _CH, _CH)
        isl = uidx_v.at[sl]
        g0 = pltpu.async_copy(c0_hbm.at[isl], b0_v.at[sl], semb)
        g1 = pltpu.async_copy(c1_hbm.at[isl], b1_v.at[sl], semb)
        g2 = pltpu.async_copy(c2_hbm.at[isl], b2_v.at[sl], semb)
        g0.wait()
        g1.wait()
        g2.wait()
        return carry
    lax.fori_loop(0, _NCH, _bgather, 0)

    sems = (sems0, sems1)
    st = [None] * _NCH

    def fire(c):
        b = c % 2
        sl = pl.ds(c * _CH, _CH)
        return (
            pltpu.async_copy(utab_hbm.at[uidx_v.at[sl]], ubuf_v.at[b], semgu),
            pltpu.async_copy(itab_hbm.at[iidx_v.at[sl]], ibuf_v.at[b], semgi),
        )

    g = {0: fire(0)}
    for c in range(_NCH):
        b = c % 2
        g[c][0].wait()
        g[c][1].wait()
        if c + 1 < _NCH:
            if c >= 1:
                st[c - 1].wait()
            g[c + 1] = fire(c + 1)

        def _mul(r, carry):
            for j in range(_K // 16):
                s = pl.ds(j * 16, 16)
                ubuf_v[b, r, s] = ubuf_v[b, r, s] * ibuf_v[b, r, s]
            return carry
        lax.fori_loop(0, _CH, _mul, 0)
        osl = pl.ds(base + c * _CH, _CH)
        st[c] = pltpu.async_copy(ubuf_v.at[b], inter_hbm.at[osl], sems[b])

    def _sel(j, carry):
        s = pl.ds(j * 16, 16)
        d = didx_v[s]
        bbuf_v[s] = jnp.where(d == 0, b0_v[s],
                              jnp.where(d == 1, b1_v[s], b2_v[s]))
        return carry
    lax.fori_loop(0, _BPW // 16, _sel, 0)
    bst = pltpu.async_copy(bbuf_v, bvals_hbm.at[pl.ds(base, _BPW)], sembst)

    st[_NCH - 2].wait()
    st[_NCH - 1].wait()
    bst.wait()


@functools.cache
def _make_sc_gather():
    return functools.partial(
        pl.kernel,
        out_type=[
            jax.ShapeDtypeStruct((_B, _K), jnp.float32),
            jax.ShapeDtypeStruct((_B,), jnp.float32),
        ],
        mesh=plsc.VectorSubcoreMesh(core_axis_name="c", subcore_axis_name="s"),
        scratch_types=[
            pltpu.VMEM((_BPW,), jnp.int32),
            pltpu.VMEM((_BPW,), jnp.int32),
            pltpu.VMEM((_BPW,), jnp.int32),
            pltpu.VMEM((_BPW,), jnp.float32),
            pltpu.VMEM((_BPW,), jnp.float32),
            pltpu.VMEM((_BPW,), jnp.float32),
            pltpu.VMEM((_BPW,), jnp.float32),
            pltpu.VMEM((2, _CH, _K), jnp.float32),
            pltpu.VMEM((2, _CH, _K), jnp.float32),
            pltpu.SemaphoreType.DMA,
            pltpu.SemaphoreType.DMA,
            pltpu.SemaphoreType.DMA,
            pltpu.SemaphoreType.DMA,
            pltpu.SemaphoreType.DMA,
            pltpu.SemaphoreType.DMA,
        ],
    )(_sc_gather_body)


_BLK = 4096


_NCODE = 120


def _tc_mlp_body(inter_ref, code_ref, bias_ref,
                 dt_ref, wk_ref, yr_ref, w1a_ref, w1t_ref, b1_ref,
                 w2_ref, b2_ref, out_ref):
    f32 = jnp.float32
    inter = inter_ref[...]
    w1t = w1t_ref[...]
    pd = jnp.dot(dt_ref[...], w1t[0:10], preferred_element_type=f32)
    pw = jnp.dot(wk_ref[...], w1t[10:20], preferred_element_type=f32)
    py = jnp.dot(yr_ref[...], w1t[20:30], preferred_element_type=f32)
    c0 = lax.broadcasted_iota(jnp.int32, (_NCODE, 1), 0)
    e3 = (c0 % 3 == lax.broadcasted_iota(jnp.int32, (_NCODE, 3), 1)).astype(f32)
    e2 = ((c0 // 3) % 2
          == lax.broadcasted_iota(jnp.int32, (_NCODE, 2), 1)).astype(f32)
    e20 = (c0 // 6
           == lax.broadcasted_iota(jnp.int32, (_NCODE, 20), 1)).astype(f32)
    p = jnp.dot(e3, pd, preferred_element_type=f32) \
        + jnp.dot(e2, pw, preferred_element_type=f32) \
        + jnp.dot(e20, py, preferred_element_type=f32)
    code = code_ref[...]
    hot_t = (jnp.broadcast_to(code, (_NCODE, _BLK))
             == lax.broadcasted_iota(jnp.int32, (_NCODE, _BLK), 0)).astype(f32)
    acc = jnp.dot(inter, w1a_ref[...], preferred_element_type=f32)
    acc = acc + lax.dot_general(hot_t, p, (((0,), (0,)), ((), ())),
                                preferred_element_type=f32)
    h = jnp.maximum(acc + b1_ref[...], 0.0)
    w2 = w2_ref[...]
    parts = [
        lax.dot_general(w2, h[k * 128:(k + 1) * 128, :],
                        (((0,), (1,)), ((), ())), preferred_element_type=f32)
        for k in range(_BLK // 128)
    ]
    out_t = jnp.concatenate(parts, axis=0)
    out_ref[...] = out_t.reshape(_BLK) + b2_ref[0, 0] + bias_ref[...]


def _tc_mlp(inter, code, bvals, dt, wk, yr, w1a, w1t, b1, w2, b2):
    grid = (_B // _BLK,)
    row_spec = pl.BlockSpec((_BLK, _K), lambda i: (i, 0))
    vec_spec = pl.BlockSpec((_BLK,), lambda i: (i,))

    def full(a):
        return pl.BlockSpec(a.shape, lambda i: tuple(0 for _ in a.shape))

    return pl.pallas_call(
        _tc_mlp_body,
        grid=grid,
        in_specs=[row_spec, vec_spec, vec_spec,
                  full(dt), full(wk), full(yr), full(w1a), full(w1t),
                  full(b1), full(w2), full(b2)],
        out_specs=vec_spec,
        out_shape=jax.ShapeDtypeStruct((_B,), jnp.float32),
    )(inter, code, bvals, dt, wk, yr, w1a, w1t, b1, w2, b2)


def kernel(user_input, item_input, daytime_input, weekend_input, year_input,
           user_table, item_table, daytime_table, weekend_table, year_table,
           user_time_bias, W1, b1, W2, b2):
    ui = user_input.astype(jnp.int32)
    di = daytime_input.astype(jnp.int32)
    inter, bvals = _make_sc_gather()(
        ui, item_input.astype(jnp.int32), di, user_table, item_table,
        user_time_bias[:, 0], user_time_bias[:, 1], user_time_bias[:, 2])
    code = di + 3 * weekend_input.astype(jnp.int32) \
        + 6 * year_input.astype(jnp.int32)
    return _tc_mlp(
        inter, code, bvals,
        daytime_table, weekend_table, year_table,
        W1[0:_K], W1[_K:], b1.reshape(1, -1), W2, b2.reshape(1, 1))


# speedup vs baseline: 1.1714x; 1.0913x over previous
import functools

import jax
import jax.numpy as jnp
from jax import lax
from jax.experimental import pallas as pl
from jax.experimental.pallas import tpu as pltpu
from jax.experimental.pallas import tpu_sc as plsc

_B = 16384
_K = 128
_NC = 2
_NS = 16
_NW = _NC * _NS
_BPW = _B // _NW
_CH = 128
_NCH = _BPW // _CH


def _sc_gather_body(uidx_hbm, iidx_hbm, didx_hbm, utab_hbm, itab_hbm,
                    c0_hbm, c1_hbm, c2_hbm,
                    inter_hbm, bvals_hbm,
                    uidx_v, iidx_v, didx_v, b0_v, b1_v, b2_v, bbuf_v,
                    ubuf_v, ibuf_v,
                    semgu, semgi, semb, sems0, sems1, sembst):
    wid = lax.axis_index("s") * _NC + lax.axis_index("c")
    base = wid * _BPW

    ldu = pltpu.async_copy(uidx_hbm.at[pl.ds(base, _BPW)], uidx_v, sems0)
    ldi = pltpu.async_copy(iidx_hbm.at[pl.ds(base, _BPW)], iidx_v, sems1)
    ldd = pltpu.async_copy(didx_hbm.at[pl.ds(base, _BPW)], didx_v, sembst)
    ldu.wait()
    ldi.wait()

    sems = (sems0, sems1)
    st = [None] * _NCH

    def fire(c):
        b = c % 2
        sl = pl.ds(c * _CH, _CH)
        return (
            pltpu.async_copy(utab_hbm.at[uidx_v.at[sl]], ubuf_v.at[b], semgu),
            pltpu.async_copy(itab_hbm.at[iidx_v.at[sl]], ibuf_v.at[b], semgi),
        )

    g = {0: fire(0)}

    bcopies = []
    for c in range(_NCH):
        sl = pl.ds(c * _CH, _CH)
        isl = uidx_v.at[sl]
        bcopies.append(pltpu.async_copy(c0_hbm.at[isl], b0_v.at[sl], semb))
        bcopies.append(pltpu.async_copy(c1_hbm.at[isl], b1_v.at[sl], semb))
        bcopies.append(pltpu.async_copy(c2_hbm.at[isl], b2_v.at[sl], semb))
    ldd.wait()
    for c in range(_NCH):
        b = c % 2
        g[c][0].wait()
        g[c][1].wait()
        if c + 1 < _NCH:
            if c >= 1:
                st[c - 1].wait()
            g[c + 1] = fire(c + 1)

        def _mul(r, carry):
            for j in range(_K // 16):
                s = pl.ds(j * 16, 16)
                ubuf_v[b, r, s] = ubuf_v[b, r, s] * ibuf_v[b, r, s]
            return carry
        lax.fori_loop(0, _CH, _mul, 0)
        osl = pl.ds(base + c * _CH, _CH)
        st[c] = pltpu.async_copy(ubuf_v.at[b], inter_hbm.at[osl], sems[b])

    for bc in bcopies:
        bc.wait()

    def _sel(j, carry):
        s = pl.ds(j * 16, 16)
        d = didx_v[s]
        bbuf_v[s] = jnp.where(d == 0, b0_v[s],
                              jnp.where(d == 1, b1_v[s], b2_v[s]))
        return carry
    lax.fori_loop(0, _BPW // 16, _sel, 0)
    bst = pltpu.async_copy(bbuf_v, bvals_hbm.at[pl.ds(base, _BPW)], sembst)

    st[_NCH - 2].wait()
    st[_NCH - 1].wait()
    bst.wait()


@functools.cache
def _make_sc_gather():
    return functools.partial(
        pl.kernel,
        out_type=[
            jax.ShapeDtypeStruct((_B, _K), jnp.float32),
            jax.ShapeDtypeStruct((_B,), jnp.float32),
        ],
        mesh=plsc.VectorSubcoreMesh(core_axis_name="c", subcore_axis_name="s"),
        scratch_types=[
            pltpu.VMEM((_BPW,), jnp.int32),
            pltpu.VMEM((_BPW,), jnp.int32),
            pltpu.VMEM((_BPW,), jnp.int32),
            pltpu.VMEM((_BPW,), jnp.float32),
            pltpu.VMEM((_BPW,), jnp.float32),
            pltpu.VMEM((_BPW,), jnp.float32),
            pltpu.VMEM((_BPW,), jnp.float32),
            pltpu.VMEM((2, _CH, _K), jnp.float32),
            pltpu.VMEM((2, _CH, _K), jnp.float32),
            pltpu.SemaphoreType.DMA,
            pltpu.SemaphoreType.DMA,
            pltpu.SemaphoreType.DMA,
            pltpu.SemaphoreType.DMA,
            pltpu.SemaphoreType.DMA,
            pltpu.SemaphoreType.DMA,
        ],
    )(_sc_gather_body)


_BLK = 4096


_NCODE = 120


def _tc_mlp_body(inter_ref, code_ref, bias_ref,
                 dt_ref, wk_ref, yr_ref, w1a_ref, w1t_ref, b1_ref,
                 w2_ref, b2_ref, out_ref):
    f32 = jnp.float32
    inter = inter_ref[...]
    w1t = w1t_ref[...]
    pd = jnp.dot(dt_ref[...], w1t[0:10], preferred_element_type=f32)
    pw = jnp.dot(wk_ref[...], w1t[10:20], preferred_element_type=f32)
    py = jnp.dot(yr_ref[...], w1t[20:30], preferred_element_type=f32)
    c0 = lax.broadcasted_iota(jnp.int32, (_NCODE, 1), 0)
    e3 = (c0 % 3 == lax.broadcasted_iota(jnp.int32, (_NCODE, 3), 1)).astype(f32)
    e2 = ((c0 // 3) % 2
          == lax.broadcasted_iota(jnp.int32, (_NCODE, 2), 1)).astype(f32)
    e20 = (c0 // 6
           == lax.broadcasted_iota(jnp.int32, (_NCODE, 20), 1)).astype(f32)
    p = jnp.dot(e3, pd, preferred_element_type=f32) \
        + jnp.dot(e2, pw, preferred_element_type=f32) \
        + jnp.dot(e20, py, preferred_element_type=f32)
    code = code_ref[...]
    hot_t = (jnp.broadcast_to(code, (_NCODE, _BLK))
             == lax.broadcasted_iota(jnp.int32, (_NCODE, _BLK), 0)).astype(f32)
    acc = jnp.dot(inter, w1a_ref[...], preferred_element_type=f32)
    acc = acc + lax.dot_general(hot_t, p, (((0,), (0,)), ((), ())),
                                preferred_element_type=f32)
    h = jnp.maximum(acc + b1_ref[...], 0.0)
    w2 = w2_ref[...]
    parts = [
        lax.dot_general(w2, h[k * 128:(k + 1) * 128, :],
                        (((0,), (1,)), ((), ())), preferred_element_type=f32)
        for k in range(_BLK // 128)
    ]
    out_t = jnp.concatenate(parts, axis=0)
    out_ref[...] = out_t.reshape(_BLK) + b2_ref[0, 0] + bias_ref[...]


def _tc_mlp(inter, code, bvals, dt, wk, yr, w1a, w1t, b1, w2, b2):
    grid = (_B // _BLK,)
    row_spec = pl.BlockSpec((_BLK, _K), lambda i: (i, 0))
    vec_spec = pl.BlockSpec((_BLK,), lambda i: (i,))

    def full(a):
        return pl.BlockSpec(a.shape, lambda i: tuple(0 for _ in a.shape))

    return pl.pallas_call(
        _tc_mlp_body,
        grid=grid,
        in_specs=[row_spec, vec_spec, vec_spec,
                  full(dt), full(wk), full(yr), full(w1a), full(w1t),
                  full(b1), full(w2), full(b2)],
        out_specs=vec_spec,
        out_shape=jax.ShapeDtypeStruct((_B,), jnp.float32),
    )(inter, code, bvals, dt, wk, yr, w1a, w1t, b1, w2, b2)


def kernel(user_input, item_input, daytime_input, weekend_input, year_input,
           user_table, item_table, daytime_table, weekend_table, year_table,
           user_time_bias, W1, b1, W2, b2):
    ui = user_input.astype(jnp.int32)
    di = daytime_input.astype(jnp.int32)
    inter, bvals = _make_sc_gather()(
        ui, item_input.astype(jnp.int32), di, user_table, item_table,
        user_time_bias[:, 0], user_time_bias[:, 1], user_time_bias[:, 2])
    code = di + 3 * weekend_input.astype(jnp.int32) \
        + 6 * year_input.astype(jnp.int32)
    return _tc_mlp(
        inter, code, bvals,
        daytime_table, weekend_table, year_table,
        W1[0:_K], W1[_K:], b1.reshape(1, -1), W2, b2.reshape(1, 1))


# speedup vs baseline: 1.1793x; 1.0067x over previous
import functools

import jax
import jax.numpy as jnp
from jax import lax
from jax.experimental import pallas as pl
from jax.experimental.pallas import tpu as pltpu
from jax.experimental.pallas import tpu_sc as plsc

_B = 16384
_K = 128
_NC = 2
_NS = 16
_NW = _NC * _NS
_BPW = _B // _NW
_CH = 128
_NCH = _BPW // _CH


def _sc_gather_body(uidx_hbm, iidx_hbm, didx_hbm, utab_hbm, itab_hbm,
                    c0_hbm, c1_hbm, c2_hbm,
                    inter_hbm, bvals_hbm,
                    uidx_v, iidx_v, didx_v, b0_v, b1_v, b2_v, bbuf_v,
                    ubuf_v, ibuf_v,
                    semgu, semgi, semb, sems0, sems1, sembst):
    wid = lax.axis_index("s") * _NC + lax.axis_index("c")
    base = wid * _BPW

    ldu = pltpu.async_copy(uidx_hbm.at[pl.ds(base, _BPW)], uidx_v, sems0)
    ldi = pltpu.async_copy(iidx_hbm.at[pl.ds(base, _BPW)], iidx_v, sems1)
    ldd = pltpu.async_copy(didx_hbm.at[pl.ds(base, _BPW)], didx_v, sembst)
    ldu.wait()
    ldi.wait()

    sems = (sems0, sems1)
    st = [None] * _NCH

    def fire(c):
        b = c % 2
        sl = pl.ds(c * _CH, _CH)
        return (
            pltpu.async_copy(utab_hbm.at[uidx_v.at[sl]], ubuf_v.at[b], semgu),
            pltpu.async_copy(itab_hbm.at[iidx_v.at[sl]], ibuf_v.at[b], semgi),
        )

    g = {0: fire(0)}

    bcopies = []
    for c in range(_NCH):
        sl = pl.ds(c * _CH, _CH)
        isl = uidx_v.at[sl]
        bcopies.append(pltpu.async_copy(c0_hbm.at[isl], b0_v.at[sl], semb))
        bcopies.append(pltpu.async_copy(c1_hbm.at[isl], b1_v.at[sl], semb))
        bcopies.append(pltpu.async_copy(c2_hbm.at[isl], b2_v.at[sl], semb))
    ldd.wait()
    for c in range(_NCH):
        b = c % 2
        g[c][0].wait()
        g[c][1].wait()
        if c + 1 < _NCH:
            if c >= 1:
                st[c - 1].wait()
            g[c + 1] = fire(c + 1)

        def _mul(r, carry):
            for j in range(_K // 16):
                s = pl.ds(j * 16, 16)
                ubuf_v[b, r, s] = ubuf_v[b, r, s] * ibuf_v[b, r, s]
            return carry
        lax.fori_loop(0, _CH, _mul, 0)
        osl = pl.ds(base + c * _CH, _CH)
        st[c] = pltpu.async_copy(ubuf_v.at[b], inter_hbm.at[osl], sems[b])

    for bc in bcopies:
        bc.wait()

    def _sel(j, carry):
        s = pl.ds(j * 16, 16)
        d = didx_v[s]
        bbuf_v[s] = jnp.where(d == 0, b0_v[s],
                              jnp.where(d == 1, b1_v[s], b2_v[s]))
        return carry
    lax.fori_loop(0, _BPW // 16, _sel, 0)
    bst = pltpu.async_copy(bbuf_v, bvals_hbm.at[pl.ds(base, _BPW)], sembst)

    st[_NCH - 2].wait()
    st[_NCH - 1].wait()
    bst.wait()


@functools.cache
def _make_sc_gather():
    return functools.partial(
        pl.kernel,
        out_type=[
            jax.ShapeDtypeStruct((_B, _K), jnp.float32),
            jax.ShapeDtypeStruct((_B,), jnp.float32),
        ],
        mesh=plsc.VectorSubcoreMesh(core_axis_name="c", subcore_axis_name="s"),
        scratch_types=[
            pltpu.VMEM((_BPW,), jnp.int32),
            pltpu.VMEM((_BPW,), jnp.int32),
            pltpu.VMEM((_BPW,), jnp.int32),
            pltpu.VMEM((_BPW,), jnp.float32),
            pltpu.VMEM((_BPW,), jnp.float32),
            pltpu.VMEM((_BPW,), jnp.float32),
            pltpu.VMEM((_BPW,), jnp.float32),
            pltpu.VMEM((2, _CH, _K), jnp.float32),
            pltpu.VMEM((2, _CH, _K), jnp.float32),
            pltpu.SemaphoreType.DMA,
            pltpu.SemaphoreType.DMA,
            pltpu.SemaphoreType.DMA,
            pltpu.SemaphoreType.DMA,
            pltpu.SemaphoreType.DMA,
            pltpu.SemaphoreType.DMA,
        ],
    )(_sc_gather_body)


_BLK = 8192


_NCODE = 120


def _tc_mlp_body(inter_ref, code_ref, bias_ref,
                 dt_ref, wk_ref, yr_ref, w1a_ref, w1t_ref, b1_ref,
                 w2_ref, b2_ref, out_ref):
    f32 = jnp.float32
    inter = inter_ref[...]
    w1t = w1t_ref[...]
    pd = jnp.dot(dt_ref[...], w1t[0:10], preferred_element_type=f32)
    pw = jnp.dot(wk_ref[...], w1t[10:20], preferred_element_type=f32)
    py = jnp.dot(yr_ref[...], w1t[20:30], preferred_element_type=f32)
    c0 = lax.broadcasted_iota(jnp.int32, (_NCODE, 1), 0)
    e3 = (c0 % 3 == lax.broadcasted_iota(jnp.int32, (_NCODE, 3), 1)).astype(f32)
    e2 = ((c0 // 3) % 2
          == lax.broadcasted_iota(jnp.int32, (_NCODE, 2), 1)).astype(f32)
    e20 = (c0 // 6
           == lax.broadcasted_iota(jnp.int32, (_NCODE, 20), 1)).astype(f32)
    p = jnp.dot(e3, pd, preferred_element_type=f32) \
        + jnp.dot(e2, pw, preferred_element_type=f32) \
        + jnp.dot(e20, py, preferred_element_type=f32)
    code = code_ref[...]
    hot_t = (jnp.broadcast_to(code, (_NCODE, _BLK))
             == lax.broadcasted_iota(jnp.int32, (_NCODE, _BLK), 0)).astype(f32)
    acc = jnp.dot(inter, w1a_ref[...], preferred_element_type=f32)
    acc = acc + lax.dot_general(hot_t, p, (((0,), (0,)), ((), ())),
                                preferred_element_type=f32)
    h = jnp.maximum(acc + b1_ref[...], 0.0)
    w2 = w2_ref[...]
    parts = [
        lax.dot_general(w2, h[k * 128:(k + 1) * 128, :],
                        (((0,), (1,)), ((), ())), preferred_element_type=f32)
        for k in range(_BLK // 128)
    ]
    out_t = jnp.concatenate(parts, axis=0)
    out_ref[...] = out_t.reshape(_BLK) + b2_ref[0, 0] + bias_ref[...]


def _tc_mlp(inter, code, bvals, dt, wk, yr, w1a, w1t, b1, w2, b2):
    grid = (_B // _BLK,)
    row_spec = pl.BlockSpec((_BLK, _K), lambda i: (i, 0))
    vec_spec = pl.BlockSpec((_BLK,), lambda i: (i,))

    def full(a):
        return pl.BlockSpec(a.shape, lambda i: tuple(0 for _ in a.shape))

    return pl.pallas_call(
        _tc_mlp_body,
        grid=grid,
        in_specs=[row_spec, vec_spec, vec_spec,
                  full(dt), full(wk), full(yr), full(w1a), full(w1t),
                  full(b1), full(w2), full(b2)],
        out_specs=vec_spec,
        out_shape=jax.ShapeDtypeStruct((_B,), jnp.float32),
    )(inter, code, bvals, dt, wk, yr, w1a, w1t, b1, w2, b2)


def kernel(user_input, item_input, daytime_input, weekend_input, year_input,
           user_table, item_table, daytime_table, weekend_table, year_table,
           user_time_bias, W1, b1, W2, b2):
    ui = user_input.astype(jnp.int32)
    di = daytime_input.astype(jnp.int32)
    inter, bvals = _make_sc_gather()(
        ui, item_input.astype(jnp.int32), di, user_table, item_table,
        user_time_bias[:, 0], user_time_bias[:, 1], user_time_bias[:, 2])
    code = di + 3 * weekend_input.astype(jnp.int32) \
        + 6 * year_input.astype(jnp.int32)
    return _tc_mlp(
        inter, code, bvals,
        daytime_table, weekend_table, year_table,
        W1[0:_K], W1[_K:], b1.reshape(1, -1), W2, b2.reshape(1, 1))


# speedup vs baseline: 1.2071x; 1.0236x over previous
import functools

import jax
import jax.numpy as jnp
from jax import lax
from jax.experimental import pallas as pl
from jax.experimental.pallas import tpu as pltpu
from jax.experimental.pallas import tpu_sc as plsc

_B = 16384
_K = 128
_NC = 2
_NS = 16
_NW = _NC * _NS
_BPW = _B // _NW
_CH = 128
_NCH = _BPW // _CH


def _sc_gather_body(uidx_hbm, iidx_hbm, didx_hbm, utab_hbm, itab_hbm,
                    c0_hbm, c1_hbm, c2_hbm,
                    inter_hbm, bvals_hbm,
                    uidx_v, iidx_v, didx_v, b0_v, b1_v, b2_v, bbuf_v,
                    ubuf_v, ibuf_v,
                    semgu, semgi, semb, sems0, sems1, sems2, sembst):
    wid = lax.axis_index("s") * _NC + lax.axis_index("c")
    base = wid * _BPW

    ldu = pltpu.async_copy(uidx_hbm.at[pl.ds(base, _BPW)], uidx_v, sems0)
    ldi = pltpu.async_copy(iidx_hbm.at[pl.ds(base, _BPW)], iidx_v, sems1)
    ldd = pltpu.async_copy(didx_hbm.at[pl.ds(base, _BPW)], didx_v, sembst)
    ldu.wait()
    ldi.wait()

    sems = (sems0, sems1, sems2)
    st = [None] * _NCH

    def fire(c):
        b = c % 3
        sl = pl.ds(c * _CH, _CH)
        return (
            pltpu.async_copy(utab_hbm.at[uidx_v.at[sl]], ubuf_v.at[b], semgu),
            pltpu.async_copy(itab_hbm.at[iidx_v.at[sl]], ibuf_v.at[b], semgi),
        )

    g = {0: fire(0)}

    bcopies = []
    for c in range(_NCH):
        sl = pl.ds(c * _CH, _CH)
        isl = uidx_v.at[sl]
        bcopies.append(pltpu.async_copy(c0_hbm.at[isl], b0_v.at[sl], semb))
        bcopies.append(pltpu.async_copy(c1_hbm.at[isl], b1_v.at[sl], semb))
        bcopies.append(pltpu.async_copy(c2_hbm.at[isl], b2_v.at[sl], semb))
    ldd.wait()
    g[1] = fire(1)
    for c in range(_NCH):
        b = c % 3
        g[c][0].wait()
        g[c][1].wait()
        if c + 2 < _NCH:
            if c >= 1:
                st[c - 1].wait()
            g[c + 2] = fire(c + 2)

        def _mul(r, carry):
            for j in range(_K // 16):
                s = pl.ds(j * 16, 16)
                ubuf_v[b, r, s] = ubuf_v[b, r, s] * ibuf_v[b, r, s]
            return carry
        lax.fori_loop(0, _CH, _mul, 0)
        osl = pl.ds(base + c * _CH, _CH)
        st[c] = pltpu.async_copy(ubuf_v.at[b], inter_hbm.at[osl], sems[b])

    for bc in bcopies:
        bc.wait()

    def _sel(j, carry):
        s = pl.ds(j * 16, 16)
        d = didx_v[s]
        bbuf_v[s] = jnp.where(d == 0, b0_v[s],
                              jnp.where(d == 1, b1_v[s], b2_v[s]))
        return carry
    lax.fori_loop(0, _BPW // 16, _sel, 0)
    bst = pltpu.async_copy(bbuf_v, bvals_hbm.at[pl.ds(base, _BPW)], sembst)

    st[_NCH - 3].wait()
    st[_NCH - 2].wait()
    st[_NCH - 1].wait()
    bst.wait()


@functools.cache
def _make_sc_gather():
    return functools.partial(
        pl.kernel,
        out_type=[
            jax.ShapeDtypeStruct((_B, _K), jnp.float32),
            jax.ShapeDtypeStruct((_B,), jnp.float32),
        ],
        mesh=plsc.VectorSubcoreMesh(core_axis_name="c", subcore_axis_name="s"),
        scratch_types=[
            pltpu.VMEM((_BPW,), jnp.int32),
            pltpu.VMEM((_BPW,), jnp.int32),
            pltpu.VMEM((_BPW,), jnp.int32),
            pltpu.VMEM((_BPW,), jnp.float32),
            pltpu.VMEM((_BPW,), jnp.float32),
            pltpu.VMEM((_BPW,), jnp.float32),
            pltpu.VMEM((_BPW,), jnp.float32),
            pltpu.VMEM((3, _CH, _K), jnp.float32),
            pltpu.VMEM((3, _CH, _K), jnp.float32),
            pltpu.SemaphoreType.DMA,
            pltpu.SemaphoreType.DMA,
            pltpu.SemaphoreType.DMA,
            pltpu.SemaphoreType.DMA,
            pltpu.SemaphoreType.DMA,
            pltpu.SemaphoreType.DMA,
            pltpu.SemaphoreType.DMA,
        ],
    )(_sc_gather_body)


_BLK = 8192


_NCODE = 120


def _tc_mlp_body(inter_ref, code_ref, bias_ref,
                 dt_ref, wk_ref, yr_ref, w1a_ref, w1t_ref, b1_ref,
                 w2_ref, b2_ref, out_ref):
    f32 = jnp.float32
    inter = inter_ref[...]
    w1t = w1t_ref[...]
    pd = jnp.dot(dt_ref[...], w1t[0:10], preferred_element_type=f32)
    pw = jnp.dot(wk_ref[...], w1t[10:20], preferred_element_type=f32)
    py = jnp.dot(yr_ref[...], w1t[20:30], preferred_element_type=f32)
    c0 = lax.broadcasted_iota(jnp.int32, (_NCODE, 1), 0)
    e3 = (c0 % 3 == lax.broadcasted_iota(jnp.int32, (_NCODE, 3), 1)).astype(f32)
    e2 = ((c0 // 3) % 2
          == lax.broadcasted_iota(jnp.int32, (_NCODE, 2), 1)).astype(f32)
    e20 = (c0 // 6
           == lax.broadcasted_iota(jnp.int32, (_NCODE, 20), 1)).astype(f32)
    p = jnp.dot(e3, pd, preferred_element_type=f32) \
        + jnp.dot(e2, pw, preferred_element_type=f32) \
        + jnp.dot(e20, py, preferred_element_type=f32)
    code = code_ref[...]
    hot_t = (jnp.broadcast_to(code, (_NCODE, _BLK))
             == lax.broadcasted_iota(jnp.int32, (_NCODE, _BLK), 0)).astype(f32)
    acc = jnp.dot(inter, w1a_ref[...], preferred_element_type=f32)
    acc = acc + lax.dot_general(hot_t, p, (((0,), (0,)), ((), ())),
                                preferred_element_type=f32)
    h = jnp.maximum(acc + b1_ref[...], 0.0)
    w2 = w2_ref[...]
    parts = [
        lax.dot_general(w2, h[k * 128:(k + 1) * 128, :],
                        (((0,), (1,)), ((), ())), preferred_element_type=f32)
        for k in range(_BLK // 128)
    ]
    out_t = jnp.concatenate(parts, axis=0)
    out_ref[...] = out_t.reshape(_BLK) + b2_ref[0, 0] + bias_ref[...]


def _tc_mlp(inter, code, bvals, dt, wk, yr, w1a, w1t, b1, w2, b2):
    grid = (_B // _BLK,)
    row_spec = pl.BlockSpec((_BLK, _K), lambda i: (i, 0))
    vec_spec = pl.BlockSpec((_BLK,), lambda i: (i,))

    def full(a):
        return pl.BlockSpec(a.shape, lambda i: tuple(0 for _ in a.shape))

    return pl.pallas_call(
        _tc_mlp_body,
        grid=grid,
        in_specs=[row_spec, vec_spec, vec_spec,
                  full(dt), full(wk), full(yr), full(w1a), full(w1t),
                  full(b1), full(w2), full(b2)],
        out_specs=vec_spec,
        out_shape=jax.ShapeDtypeStruct((_B,), jnp.float32),
    )(inter, code, bvals, dt, wk, yr, w1a, w1t, b1, w2, b2)


def kernel(user_input, item_input, daytime_input, weekend_input, year_input,
           user_table, item_table, daytime_table, weekend_table, year_table,
           user_time_bias, W1, b1, W2, b2):
    ui = user_input.astype(jnp.int32)
    di = daytime_input.astype(jnp.int32)
    inter, bvals = _make_sc_gather()(
        ui, item_input.astype(jnp.int32), di, user_table, item_table,
        user_time_bias[:, 0], user_time_bias[:, 1], user_time_bias[:, 2])
    code = di + 3 * weekend_input.astype(jnp.int32) \
        + 6 * year_input.astype(jnp.int32)
    return _tc_mlp(
        inter, code, bvals,
        daytime_table, weekend_table, year_table,
        W1[0:_K], W1[_K:], b1.reshape(1, -1), W2, b2.reshape(1, 1))
